# baseline (speedup 1.0000x reference)
import functools

import jax
import jax.numpy as jnp
from jax import lax
from jax.experimental import pallas as pl
from jax.experimental.pallas import tpu as pltpu
from jax.experimental.pallas import tpu_sc as plsc

NC = 2
NS = 16
L = 16
NW = NC * NS

f32 = jnp.float32
i32 = jnp.int32

_SC_PARAMS = pltpu.CompilerParams(use_tc_tiling_on_sc=False,
                                  needs_layout_passes=False)

_B1, _W2, _B2, _W3, _B3, _WC, _BC = 0, 4, 20, 24, 32, 34, 48


def _rsqrt16(x):
    xi = plsc.bitcast(x, i32)
    yi = jnp.int32(0x5F3759DF) - (xi >> 1)
    y = plsc.bitcast(yi, f32)
    for _ in range(3):
        y = y * (1.5 - 0.5 * x * y * y)
    return y


def _tanh16(x):
    e = jnp.exp(x * 2.0)
    return 1.0 - 2.0 / (e + 1.0)


def _make_gcn_kernel(np_, ept):
    npt = np_ // NS
    npc = np_ // L
    cpt = npc // NS
    mesh = plsc.VectorSubcoreMesh(core_axis_name="c", subcore_axis_name="s")

    @functools.partial(
        pl.kernel,
        out_type=(
            jax.ShapeDtypeStruct((np_, 7), f32),
            jax.ShapeDtypeStruct((np_, 2), f32),
        ),
        mesh=mesh,
        compiler_params=_SC_PARAMS,
        scratch_types=[
            pltpu.VMEM_SHARED((4, np_), f32),
            pltpu.VMEM_SHARED((L, 4, npc), f32),
            pltpu.VMEM((4, np_), f32),
            pltpu.VMEM((L, 4, npc), f32),
            pltpu.VMEM((ept,), i32),
            pltpu.VMEM((npt, 4), f32),
            pltpu.VMEM((npt,), f32),
            pltpu.VMEM((4, npt), f32),
            pltpu.VMEM((L, 4, cpt), f32),
            pltpu.VMEM((L, 4, cpt), f32),
            pltpu.VMEM((npt, 7), f32),
            pltpu.VMEM((npt, 2), f32),
            pltpu.VMEM((64,), f32),
            pltpu.SemaphoreType.DMA,
            pltpu.SemaphoreType.DMA,
            pltpu.SemaphoreType.DMA,
        ],
    )
    def gcn_kernel(hp1_hbm, ev_hbm, pbuf_hbm,
                   logits_hbm, h_hbm,
                   g_sh, acc_sh, g_v, acc_v, ev,
                   hp_v, dinv_v, gbuf, ibuf, abuf, lbuf, hbuf, pv,
                   sem0, sem1, sem2):
        c = lax.axis_index("c")
        s = lax.axis_index("s")
        base_n = s * npt
        csl = pl.ds(s * cpt, cpt)
        own_half = ((s < NS // 2) & (c == 0)) | ((s >= NS // 2) & (c == 1))

        cp_hp = pltpu.async_copy(hp1_hbm.at[pl.ds(base_n, npt), :], hp_v, sem0)
        cp_pv = pltpu.async_copy(pbuf_hbm, pv, sem1)
        cp_ev = pltpu.async_copy(ev_hbm.at[pl.ds(s * ept, ept)], ev, sem2)

        z = jnp.zeros((L,), f32)
        lanes = lax.iota(i32, L)
        zi = jnp.zeros((L,), i32)
        jc = [jnp.full((L,), j, i32) for j in range(4)]
        iot = lax.iota(i32, L)

        @pl.loop(0, npc // L)
        def _(i):
            sl = pl.ds(i * L, L)
            for r in range(L):
                for j in range(4):
                    acc_v[r, j, sl] = z

        @pl.loop(0, cpt // L)
        def _(i):
            sl = pl.ds(i * L, L)
            for r in range(L):
                for j in range(4):
                    ibuf[r, j, sl] = z

        pltpu.sync_copy(ibuf, acc_sh.at[:, :, csl])
        plsc.subcore_barrier()

        ones = jnp.ones((L,), f32)
        cp_ev.wait()

        @pl.loop(0, ept // L, unroll=5)
        def _(i):
            d16 = ev[pl.ds(i * L, L)] >> 16
            plsc.addupdate_scatter(acc_v, [d16 & 15, zi, d16 >> 4], ones)

        pltpu.sync_copy(acc_v, acc_sh.at[iot], add=True)
        plsc.subcore_barrier()
        pltpu.sync_copy(acc_sh.at[:, :, csl], abuf)

        cp_hp.wait()
        cp_pv.wait()
        pvecs = [pv[pl.ds(16 * t, L)] for t in range(4)]

        def _p(off):
            return pvecs[off // L][off % L]

        @pl.loop(0, npt // L)
        def _(i):
            sl = pl.ds(i * L, L)
            rows = lanes + i * L
            ifull = zi + i
            deg = plsc.load_gather(abuf, [lanes, zi, ifull]) + 1.0
            dv = _rsqrt16(deg)
            dinv_v[sl] = dv
            for j in range(4):
                gj = dv * plsc.load_gather(hp_v, [rows, jc[j]])
                gbuf[j, sl] = gj
                plsc.store_scatter(ibuf, [lanes, jc[j], ifull], gj)

        for j in range(4):
            pltpu.sync_copy(gbuf.at[j], g_sh.at[j, pl.ds(base_n, npt)])
        pltpu.sync_copy(ibuf, acc_sh.at[:, :, csl])
        plsc.subcore_barrier()

        layer_cfg = [
            (4, 4, _B1, _W2),
            (4, 2, _B2, _W3),
            (2, None, None, None),
        ]
        for (w_e, w_nxt, b_off, w_off) in layer_cfg:
            cp_g = pltpu.async_copy(g_sh, g_v, sem0)

            @pl.loop(0, npc // L)
            def _(i):
                sl = pl.ds(i * L, L)
                for r in range(L):
                    for j in range(w_e):
                        acc_v[r, j, sl] = z

            cp_g.wait()

            @pl.loop(0, ept // L, unroll=5)
            def _(i):
                e16 = ev[pl.ds(i * L, L)]
                s16 = e16 & 0xFFFF
                d16 = e16 >> 16
                dlo = d16 & 15
                dhi = d16 >> 4
                for j in range(w_e):
                    m = plsc.load_gather(g_v, [jc[j], s16])
                    plsc.addupdate_scatter(acc_v, [dlo, jc[j], dhi], m)

            pltpu.sync_copy(acc_v, acc_sh.at[iot], add=True)
            plsc.subcore_barrier()
            pltpu.sync_copy(acc_sh.at[:, :, csl], abuf)

            if w_nxt is not None:
                bias = [_p(b_off + j) for j in range(w_e)]
                wmat = [[_p(w_off + j * w_nxt + k) for k in range(w_nxt)]
                        for j in range(w_e)]

                @pl.loop(0, npt // L)
                def _(i):
                    sl = pl.ds(i * L, L)
                    ifull = zi + i
                    dv = dinv_v[sl]
                    h = []
                    for j in range(w_e):
                        a = plsc.load_gather(abuf, [lanes, jc[j], ifull])
                        h.append(_tanh16(dv * a + bias[j]))
                    for k in range(w_nxt):
                        acc = h[0] * wmat[0][k]
                        for j in range(1, w_e):
                            acc = acc + h[j] * wmat[j][k]
                        gk = dv * acc
                        gbuf[k, sl] = gk
                        plsc.store_scatter(ibuf, [lanes, jc[k], ifull], gk)

                plsc.subcore_barrier()
                for k in range(w_nxt):
                    pltpu.sync_copy(gbuf.at[k], g_sh.at[k, pl.ds(base_n, npt)])
                pltpu.sync_copy(ibuf, acc_sh.at[:, :, csl])
                plsc.subcore_barrier()

        b3 = [_p(_B3 + j) for j in range(2)]
        wc = [[_p(_WC + j * 7 + k) for k in range(7)] for j in range(2)]
        bc = [_p(_BC + k) for k in range(7)]
        kc = [jnp.full((L,), k, i32) for k in range(7)]

        @pl.loop(0, npt // L)
        def _(i):
            sl = pl.ds(i * L, L)
            rows = lanes + i * L
            ifull = zi + i
            dv = dinv_v[sl]
            h = []
            for j in range(2):
                a = plsc.load_gather(abuf, [lanes, jc[j], ifull])
                hj = _tanh16(dv * a + b3[j])
                h.append(hj)
                plsc.store_scatter(hbuf, [rows, jc[j]], hj)
            for k in range(7):
                lo = h[0] * wc[0][k] + h[1] * wc[1][k] + bc[k]
                plsc.store_scatter(lbuf, [rows, kc[k]], lo)

        @pl.when(own_half)
        def _():
            pltpu.sync_copy(lbuf, logits_hbm.at[pl.ds(base_n, npt), :])
            pltpu.sync_copy(hbuf, h_hbm.at[pl.ds(base_n, npt), :])

    return gcn_kernel


def _mm_body(x_ref, w_ref, o_ref):
    o_ref[...] = jnp.dot(x_ref[...], w_ref[...],
                         preferred_element_type=f32,
                         precision=lax.Precision.HIGHEST)


def kernel(x, edge_index, W1, b1, W2, b2, W3, b3, Wc, bc):
    n, df = x.shape
    e = edge_index.shape[1]
    np_ = ((n + NS * L - 1) // (NS * L)) * NS * L
    ep = ((e + NS * L - 1) // (NS * L)) * NS * L
    ept = ep // NS
    rows2d = np_ // 128

    xp = jnp.pad(x, ((0, np_ - n), (0, 0)))
    src = edge_index[0]
    dst = edge_index[1]
    if ep != e:
        fill = jnp.full((ep - e,), n, i32)
        src = jnp.concatenate([src, fill])
        dst = jnp.concatenate([dst, fill])
    ev = src | (dst << 16)
    pbuf = jnp.concatenate([
        b1, W2.reshape(-1), b2, W3.reshape(-1), b3, Wc.reshape(-1), bc,
    ])
    pbuf = jnp.pad(pbuf, (0, 64 - pbuf.shape[0]))

    blk = 1280
    hp1 = pl.pallas_call(
        _mm_body,
        grid=(np_ // blk,),
        in_specs=[
            pl.BlockSpec((blk, df), lambda i: (i, 0)),
            pl.BlockSpec((df, 4), lambda i: (0, 0)),
        ],
        out_specs=pl.BlockSpec((blk, 4), lambda i: (i, 0)),
        out_shape=jax.ShapeDtypeStruct((np_, 4), f32),
    )(xp, W1)

    logits_p, h_p = _make_gcn_kernel(np_, ept)(hp1, ev, pbuf)
    return (logits_p[:n], h_p[:n])


# speedup vs baseline: 1.0003x; 1.0003x over previous
import functools

import jax
import jax.numpy as jnp
from jax import lax
from jax.experimental import pallas as pl
from jax.experimental.pallas import tpu as pltpu
from jax.experimental.pallas import tpu_sc as plsc

NC = 2
NS = 16
L = 16
NW = NC * NS

f32 = jnp.float32
i32 = jnp.int32

_SC_PARAMS = pltpu.CompilerParams(use_tc_tiling_on_sc=False,
                                  needs_layout_passes=False)

_B1, _W2, _B2, _W3, _B3, _WC, _BC = 0, 4, 20, 24, 32, 34, 48


def _rsqrt16(x):
    xi = plsc.bitcast(x, i32)
    yi = jnp.int32(0x5F3759DF) - (xi >> 1)
    y = plsc.bitcast(yi, f32)
    for _ in range(3):
        y = y * (1.5 - 0.5 * x * y * y)
    return y


def _tanh16(x):
    e = jnp.exp(x * 2.0)
    return 1.0 - 2.0 / (e + 1.0)


def _make_gcn_kernel(np_, ept):
    npt = np_ // NS
    npc = np_ // L
    cpt = npc // NS
    mesh = plsc.VectorSubcoreMesh(core_axis_name="c", subcore_axis_name="s")

    @functools.partial(
        pl.kernel,
        out_type=(
            jax.ShapeDtypeStruct((np_, 7), f32),
            jax.ShapeDtypeStruct((np_, 2), f32),
        ),
        mesh=mesh,
        compiler_params=_SC_PARAMS,
        scratch_types=[
            pltpu.VMEM_SHARED((4, np_), f32),
            pltpu.VMEM_SHARED((L, 4, npc), f32),
            pltpu.VMEM((4, np_), f32),
            pltpu.VMEM((L, 4, npc), f32),
            pltpu.VMEM((ept,), i32),
            pltpu.VMEM((npt, 4), f32),
            pltpu.VMEM((npt,), f32),
            pltpu.VMEM((4, npt), f32),
            pltpu.VMEM((L, 4, cpt), f32),
            pltpu.VMEM((L, 4, cpt), f32),
            pltpu.VMEM((npt, 7), f32),
            pltpu.VMEM((npt, 2), f32),
            pltpu.VMEM((64,), f32),
            pltpu.SemaphoreType.DMA,
            pltpu.SemaphoreType.DMA,
            pltpu.SemaphoreType.DMA,
        ],
    )
    def gcn_kernel(hp1_hbm, ev_hbm, pbuf_hbm,
                   logits_hbm, h_hbm,
                   g_sh, acc_sh, g_v, acc_v, ev,
                   hp_v, dinv_v, gbuf, ibuf, abuf, lbuf, hbuf, pv,
                   sem0, sem1, sem2):
        c = lax.axis_index("c")
        s = lax.axis_index("s")
        base_n = s * npt
        csl = pl.ds(s * cpt, cpt)
        own_half = ((s < NS // 2) & (c == 0)) | ((s >= NS // 2) & (c == 1))

        cp_hp = pltpu.async_copy(hp1_hbm.at[pl.ds(base_n, npt), :], hp_v, sem0)
        cp_pv = pltpu.async_copy(pbuf_hbm, pv, sem1)
        cp_ev = pltpu.async_copy(ev_hbm.at[pl.ds(s * ept, ept)], ev, sem2)

        z = jnp.zeros((L,), f32)
        lanes = lax.iota(i32, L)
        zi = jnp.zeros((L,), i32)
        jc = [jnp.full((L,), j, i32) for j in range(4)]
        iot = lax.iota(i32, L)

        @pl.loop(0, npc // L)
        def _(i):
            sl = pl.ds(i * L, L)
            for r in range(L):
                for j in range(4):
                    acc_v[r, j, sl] = z

        @pl.loop(0, cpt // L)
        def _(i):
            sl = pl.ds(i * L, L)
            for r in range(L):
                for j in range(4):
                    ibuf[r, j, sl] = z

        pltpu.sync_copy(ibuf, acc_sh.at[:, :, csl])
        plsc.subcore_barrier()

        ones = jnp.ones((L,), f32)
        cp_ev.wait()

        with jax.named_scope("count"):
            @pl.loop(0, ept // L, unroll=5)
            def _(i):
                d16 = ev[pl.ds(i * L, L)] >> 16
                plsc.addupdate_scatter(acc_v, [d16 & 15, zi, d16 >> 4], ones)

        with jax.named_scope("cntred"):
            pltpu.sync_copy(acc_v, acc_sh.at[iot], add=True)
            plsc.subcore_barrier()
            pltpu.sync_copy(acc_sh.at[:, :, csl], abuf)

        cp_hp.wait()
        cp_pv.wait()
        pvecs = [pv[pl.ds(16 * t, L)] for t in range(4)]

        def _p(off):
            return pvecs[off // L][off % L]

        @pl.loop(0, npt // L)
        def _(i):
            sl = pl.ds(i * L, L)
            rows = lanes + i * L
            ifull = zi + i
            deg = plsc.load_gather(abuf, [lanes, zi, ifull]) + 1.0
            dv = _rsqrt16(deg)
            dinv_v[sl] = dv
            for j in range(4):
                gj = dv * plsc.load_gather(hp_v, [rows, jc[j]])
                gbuf[j, sl] = gj
                plsc.store_scatter(ibuf, [lanes, jc[j], ifull], gj)

        for j in range(4):
            pltpu.sync_copy(gbuf.at[j], g_sh.at[j, pl.ds(base_n, npt)])
        pltpu.sync_copy(ibuf, acc_sh.at[:, :, csl])
        plsc.subcore_barrier()

        layer_cfg = [
            (4, 4, _B1, _W2),
            (4, 2, _B2, _W3),
            (2, None, None, None),
        ]
        for li, (w_e, w_nxt, b_off, w_off) in enumerate(layer_cfg):
            with jax.named_scope(f"zf{li}"):
                cp_g = pltpu.async_copy(g_sh, g_v, sem0)

                @pl.loop(0, npc // L)
                def _(i):
                    sl = pl.ds(i * L, L)
                    for r in range(L):
                        for j in range(w_e):
                            acc_v[r, j, sl] = z

                cp_g.wait()

            with jax.named_scope(f"edge{li}"):
                @pl.loop(0, ept // L, unroll=5)
                def _(i):
                    e16 = ev[pl.ds(i * L, L)]
                    s16 = e16 & 0xFFFF
                    d16 = e16 >> 16
                    dlo = d16 & 15
                    dhi = d16 >> 4
                    for j in range(w_e):
                        m = plsc.load_gather(g_v, [jc[j], s16])
                        plsc.addupdate_scatter(acc_v, [dlo, jc[j], dhi], m)

            with jax.named_scope(f"red{li}"):
                pltpu.sync_copy(acc_v, acc_sh.at[iot], add=True)
                plsc.subcore_barrier()
                pltpu.sync_copy(acc_sh.at[:, :, csl], abuf)

            if w_nxt is not None:
                bias = [_p(b_off + j) for j in range(w_e)]
                wmat = [[_p(w_off + j * w_nxt + k) for k in range(w_nxt)]
                        for j in range(w_e)]

                @pl.loop(0, npt // L)
                def _(i):
                    sl = pl.ds(i * L, L)
                    ifull = zi + i
                    dv = dinv_v[sl]
                    h = []
                    for j in range(w_e):
                        a = plsc.load_gather(abuf, [lanes, jc[j], ifull])
                        h.append(_tanh16(dv * a + bias[j]))
                    for k in range(w_nxt):
                        acc = h[0] * wmat[0][k]
                        for j in range(1, w_e):
                            acc = acc + h[j] * wmat[j][k]
                        gk = dv * acc
                        gbuf[k, sl] = gk
                        plsc.store_scatter(ibuf, [lanes, jc[k], ifull], gk)

                plsc.subcore_barrier()
                for k in range(w_nxt):
                    pltpu.sync_copy(gbuf.at[k], g_sh.at[k, pl.ds(base_n, npt)])
                pltpu.sync_copy(ibuf, acc_sh.at[:, :, csl])
                plsc.subcore_barrier()

        b3 = [_p(_B3 + j) for j in range(2)]
        wc = [[_p(_WC + j * 7 + k) for k in range(7)] for j in range(2)]
        bc = [_p(_BC + k) for k in range(7)]
        kc = [jnp.full((L,), k, i32) for k in range(7)]

        @pl.loop(0, npt // L)
        def _(i):
            sl = pl.ds(i * L, L)
            rows = lanes + i * L
            ifull = zi + i
            dv = dinv_v[sl]
            h = []
            for j in range(2):
                a = plsc.load_gather(abuf, [lanes, jc[j], ifull])
                hj = _tanh16(dv * a + b3[j])
                h.append(hj)
                plsc.store_scatter(hbuf, [rows, jc[j]], hj)
            for k in range(7):
                lo = h[0] * wc[0][k] + h[1] * wc[1][k] + bc[k]
                plsc.store_scatter(lbuf, [rows, kc[k]], lo)

        @pl.when(own_half)
        def _():
            pltpu.sync_copy(lbuf, logits_hbm.at[pl.ds(base_n, npt), :])
            pltpu.sync_copy(hbuf, h_hbm.at[pl.ds(base_n, npt), :])

    return gcn_kernel


def _mm_body(x_ref, w_ref, o_ref):
    o_ref[...] = jnp.dot(x_ref[...], w_ref[...],
                         preferred_element_type=f32,
                         precision=lax.Precision.HIGHEST)


def kernel(x, edge_index, W1, b1, W2, b2, W3, b3, Wc, bc):
    n, df = x.shape
    e = edge_index.shape[1]
    np_ = ((n + NS * L - 1) // (NS * L)) * NS * L
    ep = ((e + NS * L - 1) // (NS * L)) * NS * L
    ept = ep // NS
    rows2d = np_ // 128

    xp = jnp.pad(x, ((0, np_ - n), (0, 0)))
    src = edge_index[0]
    dst = edge_index[1]
    if ep != e:
        fill = jnp.full((ep - e,), n, i32)
        src = jnp.concatenate([src, fill])
        dst = jnp.concatenate([dst, fill])
    ev = src | (dst << 16)
    pbuf = jnp.concatenate([
        b1, W2.reshape(-1), b2, W3.reshape(-1), b3, Wc.reshape(-1), bc,
    ])
    pbuf = jnp.pad(pbuf, (0, 64 - pbuf.shape[0]))

    blk = 1280
    hp1 = pl.pallas_call(
        _mm_body,
        grid=(np_ // blk,),
        in_specs=[
            pl.BlockSpec((blk, df), lambda i: (i, 0)),
            pl.BlockSpec((df, 4), lambda i: (0, 0)),
        ],
        out_specs=pl.BlockSpec((blk, 4), lambda i: (i, 0)),
        out_shape=jax.ShapeDtypeStruct((np_, 4), f32),
    )(xp, W1)

    logits_p, h_p = _make_gcn_kernel(np_, ept)(hp1, ev, pbuf)
    return (logits_p[:n], h_p[:n])


# speedup vs baseline: 1.6399x; 1.6394x over previous
import functools

import jax
import jax.numpy as jnp
from jax import lax
from jax.experimental import pallas as pl
from jax.experimental.pallas import tpu as pltpu
from jax.experimental.pallas import tpu_sc as plsc

NC = 2
NS = 16
L = 16
NW = NC * NS

f32 = jnp.float32
i32 = jnp.int32

_SC_PARAMS = pltpu.CompilerParams(use_tc_tiling_on_sc=False,
                                  needs_layout_passes=False)

_B1, _W2, _B2, _W3, _B3, _WC, _BC = 0, 4, 20, 24, 32, 34, 48


def _rsqrt16(x):
    xi = plsc.bitcast(x, i32)
    yi = jnp.int32(0x5F3759DF) - (xi >> 1)
    y = plsc.bitcast(yi, f32)
    for _ in range(3):
        y = y * (1.5 - 0.5 * x * y * y)
    return y


def _tanh16(x):
    e = jnp.exp(x * 2.0)
    return 1.0 - 2.0 / (e + 1.0)


def _make_gcn_kernel(np_, ept):
    npt = np_ // NS
    npc = np_ // L
    cpt = npc // NS
    mesh = plsc.VectorSubcoreMesh(core_axis_name="c", subcore_axis_name="s")

    @functools.partial(
        pl.kernel,
        out_type=(
            jax.ShapeDtypeStruct((np_, 7), f32),
            jax.ShapeDtypeStruct((np_, 2), f32),
        ),
        mesh=mesh,
        compiler_params=_SC_PARAMS,
        scratch_types=[
            pltpu.VMEM_SHARED((4, np_), f32),
            pltpu.VMEM_SHARED((L, 4, npc), f32),
            pltpu.VMEM((4, np_), f32),
            pltpu.VMEM((L, 4, npc), f32),
            pltpu.VMEM((ept,), i32),
            pltpu.VMEM((npt, 4), f32),
            pltpu.VMEM((npt,), f32),
            pltpu.VMEM((4, npt), f32),
            pltpu.VMEM((L, 4, cpt), f32),
            pltpu.VMEM((L, 4, cpt), f32),
            pltpu.VMEM((npt, 7), f32),
            pltpu.VMEM((npt, 2), f32),
            pltpu.VMEM((64,), f32),
            pltpu.SemaphoreType.DMA,
            pltpu.SemaphoreType.DMA,
            pltpu.SemaphoreType.DMA,
        ],
    )
    def gcn_kernel(hp1_hbm, ev_hbm, pbuf_hbm,
                   logits_hbm, h_hbm,
                   g_sh, acc_sh, g_v, acc_v, ev,
                   hp_v, dinv_v, gbuf, ibuf, abuf, lbuf, hbuf, pv,
                   sem0, sem1, sem2):
        c = lax.axis_index("c")
        s = lax.axis_index("s")
        base_n = s * npt
        csl = pl.ds(s * cpt, cpt)
        own_half = ((s < NS // 2) & (c == 0)) | ((s >= NS // 2) & (c == 1))

        cp_hp = pltpu.async_copy(hp1_hbm.at[pl.ds(base_n, npt), :], hp_v, sem0)
        cp_pv = pltpu.async_copy(pbuf_hbm, pv, sem1)
        cp_ev = pltpu.async_copy(ev_hbm.at[pl.ds(s * ept, ept)], ev, sem2)

        z = jnp.zeros((L,), f32)
        lanes = lax.iota(i32, L)
        zi = jnp.zeros((L,), i32)
        jc = [jnp.full((L,), j, i32) for j in range(4)]
        iot = lax.iota(i32, L)

        @pl.loop(0, npc // L)
        def _(i):
            sl = pl.ds(i * L, L)
            for r in range(L):
                for j in range(4):
                    acc_v[r, j, sl] = z

        @pl.loop(0, cpt // L)
        def _(i):
            sl = pl.ds(i * L, L)
            for r in range(L):
                for j in range(4):
                    ibuf[r, j, sl] = z

        pltpu.sync_copy(ibuf, acc_sh.at[:, :, csl])
        plsc.subcore_barrier()

        ones = jnp.ones((L,), f32)
        cp_ev.wait()

        with jax.named_scope("count"):
            @plsc.parallel_loop(0, ept // L, unroll=5)
            def _(i):
                e16 = ev[pl.ds(i * L, L)]
                plsc.addupdate_scatter(acc_v, [(e16 >> 14) & 15, zi, e16 >> 18],
                                       ones)

        with jax.named_scope("cntred"):
            pltpu.sync_copy(acc_v, acc_sh.at[iot], add=True)
            plsc.subcore_barrier()
            pltpu.sync_copy(acc_sh.at[:, :, csl], abuf)

        cp_hp.wait()
        cp_pv.wait()
        pvecs = [pv[pl.ds(16 * t, L)] for t in range(4)]

        def _p(off):
            return pvecs[off // L][off % L]

        @pl.loop(0, npt // L)
        def _(i):
            sl = pl.ds(i * L, L)
            rows = lanes + i * L
            ifull = zi + i
            deg = plsc.load_gather(abuf, [lanes, zi, ifull]) + 1.0
            dv = _rsqrt16(deg)
            dinv_v[sl] = dv
            for j in range(4):
                gj = dv * plsc.load_gather(hp_v, [rows, jc[j]])
                gbuf[j, sl] = gj
                plsc.store_scatter(ibuf, [lanes, jc[j], ifull], gj)

        for j in range(4):
            pltpu.sync_copy(gbuf.at[j], g_sh.at[j, pl.ds(base_n, npt)])
        pltpu.sync_copy(ibuf, acc_sh.at[:, :, csl])
        plsc.subcore_barrier()

        layer_cfg = [
            (4, 4, _B1, _W2),
            (4, 2, _B2, _W3),
            (2, None, None, None),
        ]
        for li, (w_e, w_nxt, b_off, w_off) in enumerate(layer_cfg):
            with jax.named_scope(f"zf{li}"):
                cp_g = pltpu.async_copy(g_sh, g_v, sem0)

                @pl.loop(0, npc // L)
                def _(i):
                    sl = pl.ds(i * L, L)
                    for r in range(L):
                        for j in range(w_e):
                            acc_v[r, j, sl] = z

                cp_g.wait()

            with jax.named_scope(f"edge{li}"):
                @plsc.parallel_loop(0, ept // L, unroll=5)
                def _(i):
                    e16 = ev[pl.ds(i * L, L)]
                    s16 = e16 & 0x3FFF
                    dlo = (e16 >> 14) & 15
                    dhi = e16 >> 18
                    ms = [plsc.load_gather(g_v, [jc[j], s16])
                          for j in range(w_e)]
                    for j in range(w_e):
                        plsc.addupdate_scatter(acc_v, [dlo, jc[j], dhi], ms[j])

            with jax.named_scope(f"red{li}"):
                pltpu.sync_copy(acc_v, acc_sh.at[iot], add=True)
                plsc.subcore_barrier()
                pltpu.sync_copy(acc_sh.at[:, :, csl], abuf)

            if w_nxt is not None:
                bias = [_p(b_off + j) for j in range(w_e)]
                wmat = [[_p(w_off + j * w_nxt + k) for k in range(w_nxt)]
                        for j in range(w_e)]

                @pl.loop(0, npt // L)
                def _(i):
                    sl = pl.ds(i * L, L)
                    ifull = zi + i
                    dv = dinv_v[sl]
                    h = []
                    for j in range(w_e):
                        a = plsc.load_gather(abuf, [lanes, jc[j], ifull])
                        h.append(_tanh16(dv * a + bias[j]))
                    for k in range(w_nxt):
                        acc = h[0] * wmat[0][k]
                        for j in range(1, w_e):
                            acc = acc + h[j] * wmat[j][k]
                        gk = dv * acc
                        gbuf[k, sl] = gk
                        plsc.store_scatter(ibuf, [lanes, jc[k], ifull], gk)

                plsc.subcore_barrier()
                for k in range(w_nxt):
                    pltpu.sync_copy(gbuf.at[k], g_sh.at[k, pl.ds(base_n, npt)])
                pltpu.sync_copy(ibuf, acc_sh.at[:, :, csl])
                plsc.subcore_barrier()

        b3 = [_p(_B3 + j) for j in range(2)]
        wc = [[_p(_WC + j * 7 + k) for k in range(7)] for j in range(2)]
        bc = [_p(_BC + k) for k in range(7)]
        kc = [jnp.full((L,), k, i32) for k in range(7)]

        @pl.loop(0, npt // L)
        def _(i):
            sl = pl.ds(i * L, L)
            rows = lanes + i * L
            ifull = zi + i
            dv = dinv_v[sl]
            h = []
            for j in range(2):
                a = plsc.load_gather(abuf, [lanes, jc[j], ifull])
                hj = _tanh16(dv * a + b3[j])
                h.append(hj)
                plsc.store_scatter(hbuf, [rows, jc[j]], hj)
            for k in range(7):
                lo = h[0] * wc[0][k] + h[1] * wc[1][k] + bc[k]
                plsc.store_scatter(lbuf, [rows, kc[k]], lo)

        @pl.when(own_half)
        def _():
            pltpu.sync_copy(lbuf, logits_hbm.at[pl.ds(base_n, npt), :])
            pltpu.sync_copy(hbuf, h_hbm.at[pl.ds(base_n, npt), :])

    return gcn_kernel


def _mm_body(x_ref, w_ref, o_ref):
    o_ref[...] = jnp.dot(x_ref[...], w_ref[...],
                         preferred_element_type=f32)


def _pack_body(e_ref, o_ref):
    sv = e_ref[0]
    dv = e_ref[1]
    o_ref[...] = sv | ((dv & 15) << 14) | ((dv >> 4) << 18)


def kernel(x, edge_index, W1, b1, W2, b2, W3, b3, Wc, bc):
    n, df = x.shape
    e = edge_index.shape[1]
    np_ = ((n + NS * L - 1) // (NS * L)) * NS * L
    ep = ((e + NS * L - 1) // (NS * L)) * NS * L
    ept = ep // NS
    rows2d = np_ // 128

    ei = edge_index
    if ep != e:
        ei = jnp.concatenate(
            [ei, jnp.full((2, ep - e), n, i32)], axis=1)
    pbuf = jnp.concatenate([
        b1, W2.reshape(-1), b2, W3.reshape(-1), b3, Wc.reshape(-1), bc,
    ])
    pbuf = jnp.pad(pbuf, (0, 64 - pbuf.shape[0]))

    ev = pl.pallas_call(
        _pack_body,
        out_shape=jax.ShapeDtypeStruct((ep // 128, 128), i32),
    )(ei.reshape(2, ep // 128, 128)).reshape(ep)

    blk = 2000 if n % 2000 == 0 else n
    hp1 = pl.pallas_call(
        _mm_body,
        grid=(n // blk,),
        in_specs=[
            pl.BlockSpec((blk, df), lambda i: (i, 0)),
            pl.BlockSpec((df, 4), lambda i: (0, 0)),
        ],
        out_specs=pl.BlockSpec((blk, 4), lambda i: (i, 0)),
        out_shape=jax.ShapeDtypeStruct((n, 4), f32),
    )(x, W1)
    hp1 = jnp.pad(hp1, ((0, np_ - n), (0, 0)))

    logits_p, h_p = _make_gcn_kernel(np_, ept)(hp1, ev, pbuf)
    return (logits_p[:n], h_p[:n])
